# baseline (speedup 1.0000x reference)
import functools

import jax
import jax.numpy as jnp
from jax import lax
from jax.experimental import pallas as pl
from jax.experimental.pallas import tpu as pltpu
from jax.experimental.pallas import tpu_sc as plsc

_NC = 2
_NS = 16
_NW = _NC * _NS

_SUB_ROWS = 16


def kernel(inputs, embeddings):
    b, s, d = inputs.shape
    rows_per_w = s // _NW
    n_sub = rows_per_w // _SUB_ROWS
    n_vec = _SUB_ROWS * d // 16

    mesh = plsc.VectorSubcoreMesh(core_axis_name="c", subcore_axis_name="s")

    @functools.partial(
        pl.kernel,
        out_type=jax.ShapeDtypeStruct((b, s, d), jnp.float32),
        mesh=mesh,
        scratch_types=[
            pltpu.VMEM((_SUB_ROWS, d), jnp.float32),
            pltpu.VMEM((_SUB_ROWS, d), jnp.float32),
        ],
    )
    def sc_add(x_hbm, e_hbm, o_hbm, e_buf, x_buf):
        wid = lax.axis_index("s") * _NC + lax.axis_index("c")
        s0 = wid * rows_per_w

        def sub_body(sub, _):
            r0 = s0 + sub * _SUB_ROWS
            pltpu.sync_copy(e_hbm.at[pl.ds(r0, _SUB_ROWS), :], e_buf)
            for bb in range(b):
                pltpu.sync_copy(x_hbm.at[bb, pl.ds(r0, _SUB_ROWS), :], x_buf)

                def add_body(i, _):
                    r = lax.shift_right_logical(i, 6)
                    c = pl.multiple_of(lax.shift_left(lax.bitwise_and(i, 63), 4), 16)
                    x_buf[r, pl.ds(c, 16)] = (
                        x_buf[r, pl.ds(c, 16)] + e_buf[r, pl.ds(c, 16)]
                    )
                    return 0

                lax.fori_loop(0, n_vec, add_body, 0)
                pltpu.sync_copy(x_buf, o_hbm.at[bb, pl.ds(r0, _SUB_ROWS), :])
            return 0

        lax.fori_loop(0, n_sub, sub_body, 0)

    return sc_add(inputs, embeddings)


# speedup vs baseline: 4.0338x; 4.0338x over previous
import jax
import jax.numpy as jnp
from jax.experimental import pallas as pl

_IN_S = 512
_OUT_S = 256


def _add_kernel(x_ref, e_ref, o_ref):
    i = pl.program_id(0)
    half = (i % 2) * _OUT_S
    o_ref[...] = (
        x_ref[:, pl.ds(half, _OUT_S), :] + e_ref[pl.ds(half, _OUT_S), :][None, :, :]
    )


def kernel(inputs, embeddings):
    b, s, d = inputs.shape
    grid = (s // _OUT_S,)
    return pl.pallas_call(
        _add_kernel,
        grid=grid,
        in_specs=[
            pl.BlockSpec((b, _IN_S, d), lambda i: (0, i // 2, 0)),
            pl.BlockSpec((_IN_S, d), lambda i: (i // 2, 0)),
        ],
        out_specs=pl.BlockSpec((b, _OUT_S, d), lambda i: (0, i, 0)),
        out_shape=jax.ShapeDtypeStruct((b, s, d), inputs.dtype),
    )(inputs, embeddings)


# speedup vs baseline: 5.0352x; 1.2483x over previous
import jax
import jax.numpy as jnp
from jax.experimental import pallas as pl

_IN_S = 256
_OUT_S = 512


def _add_kernel(x_ref, e_ref, o_ref):
    i = pl.program_id(0)
    half = (i % 2) * _IN_S
    o_ref[:, pl.ds(half, _IN_S), :] = x_ref[...] + e_ref[...][None, :, :]


def kernel(inputs, embeddings):
    b, s, d = inputs.shape
    grid = (s // _IN_S,)
    return pl.pallas_call(
        _add_kernel,
        grid=grid,
        in_specs=[
            pl.BlockSpec((b, _IN_S, d), lambda i: (0, i, 0)),
            pl.BlockSpec((_IN_S, d), lambda i: (i, 0)),
        ],
        out_specs=pl.BlockSpec((b, _OUT_S, d), lambda i: (0, i // 2, 0)),
        out_shape=jax.ShapeDtypeStruct((b, s, d), inputs.dtype),
    )(inputs, embeddings)


# speedup vs baseline: 5.0944x; 1.0118x over previous
import jax
import jax.numpy as jnp
from jax.experimental import pallas as pl

_BLOCK_S = 512


def _add_kernel(x_ref, e_ref, o_ref):
    o_ref[...] = x_ref[...] + e_ref[...][None, :, :]


def kernel(inputs, embeddings):
    b, s, d = inputs.shape
    grid = (s // _BLOCK_S,)
    return pl.pallas_call(
        _add_kernel,
        grid=grid,
        in_specs=[
            pl.BlockSpec((b, _BLOCK_S, d), lambda i: (0, i, 0)),
            pl.BlockSpec((_BLOCK_S, d), lambda i: (i, 0)),
        ],
        out_specs=pl.BlockSpec((b, _BLOCK_S, d), lambda i: (0, i, 0)),
        out_shape=jax.ShapeDtypeStruct((b, s, d), inputs.dtype),
    )(inputs, embeddings)
